# Initial kernel scaffold; baseline (speedup 1.0000x reference)
import jax
import jax.numpy as jnp
from jax.experimental import pallas as pl


def kernel(h, edge_index, edge_w, W, b):
    raise NotImplementedError("write your pallas kernel here")


# speedup vs baseline: 1.6628x; 1.6628x over previous
import functools

import jax
import jax.numpy as jnp
from jax import lax
from jax.experimental import pallas as pl
from jax.experimental.pallas import tpu as pltpu
from jax.experimental.pallas import tpu_sc as plsc

N_NODES = 10000
N_EDGES = 160000
D = 256
NC = 2
NS = 16
NW = NC * NS
L = 16
Z = 312
ZLAST = N_NODES - Z * (NW - 1)
EC = 3200
C = 64
SELCAP = (C - 1) + EC + 2 * L

_mesh = plsc.VectorSubcoreMesh(
    core_axis_name="c", subcore_axis_name="s", num_cores=NC, num_subcores=NS
)


@functools.partial(
    pl.kernel,
    out_type=jax.ShapeDtypeStruct((N_NODES, D), jnp.float32),
    mesh=_mesh,
    compiler_params=pltpu.CompilerParams(needs_layout_passes=False),
    scratch_types=[
        pltpu.VMEM((EC,), jnp.int32),
        pltpu.VMEM((EC,), jnp.int32),
        pltpu.VMEM((EC,), jnp.float32),
        pltpu.VMEM((SELCAP,), jnp.int32),
        pltpu.VMEM((SELCAP,), jnp.int32),
        pltpu.VMEM((SELCAP,), jnp.float32),
        pltpu.VMEM((C, D), jnp.float32),
        pltpu.VMEM((C,), jnp.int32),
        pltpu.VMEM((ZLAST, D), jnp.float32),
    ],
)
def _sc_agg(h_hbm, ei_hbm, ew_hbm, agg_hbm,
            srcv, dstv, wvv, src_sel, dst_sel, w_sel,
            rows_v, src_idx, acc):
    cid = lax.axis_index("c")
    sid = lax.axis_index("s")
    w = sid * NC + cid
    is_last = w == NW - 1
    wlo = w * Z
    nrows = jnp.where(is_last, ZLAST, Z)
    whi = wlo + nrows
    zero16 = jnp.zeros((L,), jnp.float32)

    @pl.loop(0, ZLAST)
    def _(r):
        for k in range(D // L):
            acc[r, pl.ds(k * L, L)] = zero16

    def drain(dbase):
        for k in range(C // L):
            src_idx[pl.ds(k * L, L)] = src_sel[pl.ds(dbase + k * L, L)]
        pltpu.sync_copy(h_hbm.at[src_idx], rows_v)

        @pl.loop(0, C)
        def _(r):
            pos16 = jnp.full((L,), dbase + r, jnp.int32)
            wb = plsc.load_gather(w_sel, [pos16])
            dl = plsc.load_gather(dst_sel, [pos16])[0]
            for k in range(D // L):
                sl = pl.ds(k * L, L)
                plsc.addupdate(acc.at[dl, sl], rows_v[r, sl] * wb)

    def scan_chunk(ch, cnt):
        e0 = ch * EC
        pltpu.sync_copy(ei_hbm.at[pl.ds(e0, EC)], srcv)
        pltpu.sync_copy(ei_hbm.at[pl.ds(N_EDGES + e0, EC)], dstv)
        pltpu.sync_copy(ew_hbm.at[pl.ds(e0, EC)], wvv)

        def inner(i, cnt):
            dvec = dstv[pl.ds(i * L, L)]
            svec = srcv[pl.ds(i * L, L)]
            wvec = wvv[pl.ds(i * L, L)]
            m = (dvec >= wlo) & (dvec < whi)
            plsc.store_compressed(dst_sel.at[pl.ds(cnt, L)], dvec - wlo, mask=m)
            plsc.store_compressed(src_sel.at[pl.ds(cnt, L)], svec, mask=m)
            plsc.store_compressed(w_sel.at[pl.ds(cnt, L)], wvec, mask=m)
            return cnt + jnp.sum(m.astype(jnp.int32))

        cnt = lax.fori_loop(0, EC // L, inner, cnt)

        nd = cnt // C

        def dr(d, carry):
            drain(d * C)
            return carry

        lax.fori_loop(0, nd, dr, jnp.int32(0))
        off = nd * C
        for k in range(C // L):
            src_sel[pl.ds(k * L, L)] = src_sel[pl.ds(off + k * L, L)]
            dst_sel[pl.ds(k * L, L)] = dst_sel[pl.ds(off + k * L, L)]
            w_sel[pl.ds(k * L, L)] = w_sel[pl.ds(off + k * L, L)]
        return cnt - off

    cnt = lax.fori_loop(0, N_EDGES // EC, scan_chunk, jnp.int32(0))

    base = (cnt // L) * L
    lane = lax.iota(jnp.int32, L)
    for t in range(C // L + 1):
        off = base + t * L
        keep = (off + lane) < cnt
        src_sel[pl.ds(off, L)] = jnp.where(keep, src_sel[pl.ds(off, L)], 0)
        dst_sel[pl.ds(off, L)] = jnp.where(keep, dst_sel[pl.ds(off, L)], 0)
        w_sel[pl.ds(off, L)] = jnp.where(keep, w_sel[pl.ds(off, L)], 0.0)

    @pl.when(cnt > 0)
    def _():
        drain(0)

    @pl.when(jnp.logical_not(is_last))
    def _():
        pltpu.sync_copy(acc.at[pl.ds(0, Z)], agg_hbm.at[pl.ds(wlo, Z)])

    @pl.when(is_last)
    def _():
        pltpu.sync_copy(acc.at[pl.ds(0, ZLAST)],
                        agg_hbm.at[pl.ds((NW - 1) * Z, ZLAST)])


def _tc_body(h_ref, agg_ref, w_ref, b_ref, o_ref):
    u = jnp.dot(agg_ref[...], w_ref[...], preferred_element_type=jnp.float32)
    u = u + b_ref[...]
    g = 0.5 * u * (1.0 + lax.erf(u * 0.7071067811865476))
    o_ref[...] = h_ref[...] + g


_R = 2000


def _tc_update(h, agg, W, b2):
    return pl.pallas_call(
        _tc_body,
        grid=(N_NODES // _R,),
        in_specs=[
            pl.BlockSpec((_R, D), lambda i: (i, 0)),
            pl.BlockSpec((_R, D), lambda i: (i, 0)),
            pl.BlockSpec((D, D), lambda i: (0, 0)),
            pl.BlockSpec((1, D), lambda i: (0, 0)),
        ],
        out_specs=pl.BlockSpec((_R, D), lambda i: (i, 0)),
        out_shape=jax.ShapeDtypeStruct((N_NODES, D), jnp.float32),
    )(h, agg, W, b2)


def kernel(h, edge_index, edge_w, W, b):
    agg = _sc_agg(h, edge_index.reshape(2 * N_EDGES), edge_w)
    return _tc_update(h, agg, W, b.reshape(1, D))


# speedup vs baseline: 1.9004x; 1.1429x over previous
import functools

import jax
import jax.numpy as jnp
from jax import lax
from jax.experimental import pallas as pl
from jax.experimental.pallas import tpu as pltpu
from jax.experimental.pallas import tpu_sc as plsc

N_NODES = 10000
N_EDGES = 160000
D = 256
NC = 2
NS = 16
NW = NC * NS
L = 16
Z = 312
ZLAST = N_NODES - Z * (NW - 1)
EC = 2000
NCHUNKS = N_EDGES // EC
C = 64
SELCAP = (C - 1) + EC + 2 * L

_mesh = plsc.VectorSubcoreMesh(
    core_axis_name="c", subcore_axis_name="s", num_cores=NC, num_subcores=NS
)


@functools.partial(
    pl.kernel,
    out_type=jax.ShapeDtypeStruct((N_NODES, D), jnp.float32),
    mesh=_mesh,
    compiler_params=pltpu.CompilerParams(needs_layout_passes=False),
    scratch_types=[
        pltpu.VMEM((EC,), jnp.int32),
        pltpu.VMEM((EC,), jnp.int32),
        pltpu.VMEM((EC,), jnp.float32),
        pltpu.VMEM((EC,), jnp.int32),
        pltpu.VMEM((EC,), jnp.int32),
        pltpu.VMEM((EC,), jnp.float32),
        pltpu.VMEM((SELCAP,), jnp.int32),
        pltpu.VMEM((SELCAP,), jnp.int32),
        pltpu.VMEM((SELCAP,), jnp.float32),
        pltpu.VMEM((C, D), jnp.float32),
        pltpu.VMEM((C,), jnp.int32),
        pltpu.VMEM((ZLAST, D), jnp.float32),
        pltpu.SemaphoreType.DMA,
        pltpu.SemaphoreType.DMA,
    ],
)
def _sc_agg(h_hbm, ei_hbm, ew_hbm, agg_hbm,
            srcv_a, dstv_a, wvv_a, srcv_b, dstv_b, wvv_b,
            src_sel, dst_sel, w_sel,
            rows_v, src_idx, acc, sem_a, sem_b):
    cid = lax.axis_index("c")
    sid = lax.axis_index("s")
    w = sid * NC + cid
    is_last = w == NW - 1
    wlo = w * Z
    nrows = jnp.where(is_last, ZLAST, Z)
    whi = wlo + nrows
    zero16 = jnp.zeros((L,), jnp.float32)

    @pl.loop(0, ZLAST)
    def _(r):
        for k in range(D // L):
            acc[r, pl.ds(k * L, L)] = zero16

    def drain(dbase):
        for k in range(C // L):
            src_idx[pl.ds(k * L, L)] = src_sel[pl.ds(dbase + k * L, L)]
        pltpu.sync_copy(h_hbm.at[src_idx], rows_v)

        @pl.loop(0, C)
        def _(r):
            pos16 = jnp.full((L,), dbase + r, jnp.int32)
            wb = plsc.load_gather(w_sel, [pos16])
            dl = plsc.load_gather(dst_sel, [pos16])[0]
            for k in range(D // L):
                sl = pl.ds(k * L, L)
                plsc.addupdate(acc.at[dl, sl], rows_v[r, sl] * wb)

    def start_fetch(ch, sv, dv, wv, sem):
        e0 = ch * EC
        pltpu.async_copy(ei_hbm.at[pl.ds(e0, EC)], sv, sem)
        pltpu.async_copy(ei_hbm.at[pl.ds(N_EDGES + e0, EC)], dv, sem)
        pltpu.async_copy(ew_hbm.at[pl.ds(e0, EC)], wv, sem)

    def wait_fetch(sv, dv, wv, sem):
        pltpu.make_async_copy(ei_hbm.at[pl.ds(0, EC)], sv, sem).wait()
        pltpu.make_async_copy(ei_hbm.at[pl.ds(0, EC)], dv, sem).wait()
        pltpu.make_async_copy(ew_hbm.at[pl.ds(0, EC)], wv, sem).wait()

    def compact_drain(srcv, dstv, wvv, cnt):
        def inner(i, cnt):
            dvec = dstv[pl.ds(i * L, L)]
            svec = srcv[pl.ds(i * L, L)]
            wvec = wvv[pl.ds(i * L, L)]
            m = (dvec >= wlo) & (dvec < whi)
            plsc.store_compressed(dst_sel.at[pl.ds(cnt, L)], dvec - wlo, mask=m)
            plsc.store_compressed(src_sel.at[pl.ds(cnt, L)], svec, mask=m)
            plsc.store_compressed(w_sel.at[pl.ds(cnt, L)], wvec, mask=m)
            return cnt + jnp.sum(m.astype(jnp.int32))

        cnt = lax.fori_loop(0, EC // L, inner, cnt)

        nd = cnt // C

        def dr(d, carry):
            drain(d * C)
            return carry

        lax.fori_loop(0, nd, dr, jnp.int32(0))
        off = nd * C
        for k in range(C // L):
            src_sel[pl.ds(k * L, L)] = src_sel[pl.ds(off + k * L, L)]
            dst_sel[pl.ds(k * L, L)] = dst_sel[pl.ds(off + k * L, L)]
            w_sel[pl.ds(k * L, L)] = w_sel[pl.ds(off + k * L, L)]
        return cnt - off

    start_fetch(0, srcv_a, dstv_a, wvv_a, sem_a)

    def two_chunks(p, cnt):
        ch = p * 2
        start_fetch(ch + 1, srcv_b, dstv_b, wvv_b, sem_b)
        wait_fetch(srcv_a, dstv_a, wvv_a, sem_a)
        cnt = compact_drain(srcv_a, dstv_a, wvv_a, cnt)

        @pl.when(ch + 2 < NCHUNKS)
        def _():
            start_fetch(ch + 2, srcv_a, dstv_a, wvv_a, sem_a)

        wait_fetch(srcv_b, dstv_b, wvv_b, sem_b)
        cnt = compact_drain(srcv_b, dstv_b, wvv_b, cnt)
        return cnt

    cnt = lax.fori_loop(0, NCHUNKS // 2, two_chunks, jnp.int32(0))

    base = (cnt // L) * L
    lane = lax.iota(jnp.int32, L)
    for t in range(C // L + 1):
        off = base + t * L
        keep = (off + lane) < cnt
        src_sel[pl.ds(off, L)] = jnp.where(keep, src_sel[pl.ds(off, L)], 0)
        dst_sel[pl.ds(off, L)] = jnp.where(keep, dst_sel[pl.ds(off, L)], 0)
        w_sel[pl.ds(off, L)] = jnp.where(keep, w_sel[pl.ds(off, L)], 0.0)

    @pl.when(cnt > 0)
    def _():
        drain(0)

    @pl.when(jnp.logical_not(is_last))
    def _():
        pltpu.sync_copy(acc.at[pl.ds(0, Z)], agg_hbm.at[pl.ds(wlo, Z)])

    @pl.when(is_last)
    def _():
        pltpu.sync_copy(acc.at[pl.ds(0, ZLAST)],
                        agg_hbm.at[pl.ds((NW - 1) * Z, ZLAST)])


def _tc_body(h_ref, agg_ref, w_ref, b_ref, o_ref):
    u = jnp.dot(agg_ref[...], w_ref[...], preferred_element_type=jnp.float32)
    u = u + b_ref[...]
    g = 0.5 * u * (1.0 + lax.erf(u * 0.7071067811865476))
    o_ref[...] = h_ref[...] + g


_R = 2000


def _tc_update(h, agg, W, b2):
    return pl.pallas_call(
        _tc_body,
        grid=(N_NODES // _R,),
        in_specs=[
            pl.BlockSpec((_R, D), lambda i: (i, 0)),
            pl.BlockSpec((_R, D), lambda i: (i, 0)),
            pl.BlockSpec((D, D), lambda i: (0, 0)),
            pl.BlockSpec((1, D), lambda i: (0, 0)),
        ],
        out_specs=pl.BlockSpec((_R, D), lambda i: (i, 0)),
        out_shape=jax.ShapeDtypeStruct((N_NODES, D), jnp.float32),
    )(h, agg, W, b2)


def kernel(h, edge_index, edge_w, W, b):
    agg = _sc_agg(h, edge_index.reshape(2 * N_EDGES), edge_w)
    return _tc_update(h, agg, W, b.reshape(1, D))


# speedup vs baseline: 3.3916x; 1.7846x over previous
import functools

import jax
import jax.numpy as jnp
from jax import lax
from jax.experimental import pallas as pl
from jax.experimental.pallas import tpu as pltpu
from jax.experimental.pallas import tpu_sc as plsc

N_NODES = 10000
N_EDGES = 160000
D = 256
NC = 2
NS = 16
NW = NC * NS
L = 16
Z = 312
ZLAST = N_NODES - Z * (NW - 1)
EC = 2000
NCHUNKS = N_EDGES // EC
C = 64
SELCAP = (C - 1) + EC + 2 * L

_mesh = plsc.VectorSubcoreMesh(
    core_axis_name="c", subcore_axis_name="s", num_cores=NC, num_subcores=NS
)


@functools.partial(
    pl.kernel,
    out_type=jax.ShapeDtypeStruct((N_NODES, D), jnp.float32),
    mesh=_mesh,
    compiler_params=pltpu.CompilerParams(needs_layout_passes=False),
    scratch_types=[
        pltpu.VMEM((EC,), jnp.int32),
        pltpu.VMEM((EC,), jnp.int32),
        pltpu.VMEM((EC,), jnp.float32),
        pltpu.VMEM((EC,), jnp.int32),
        pltpu.VMEM((EC,), jnp.int32),
        pltpu.VMEM((EC,), jnp.float32),
        pltpu.VMEM((SELCAP,), jnp.int32),
        pltpu.VMEM((SELCAP,), jnp.int32),
        pltpu.VMEM((SELCAP,), jnp.float32),
        pltpu.VMEM((C, D), jnp.float32),
        pltpu.VMEM((C,), jnp.int32),
        pltpu.VMEM((ZLAST, D), jnp.float32),
        pltpu.SemaphoreType.DMA,
        pltpu.SemaphoreType.DMA,
    ],
)
def _sc_agg(h_hbm, ei_hbm, ew_hbm, agg_hbm,
            srcv_a, dstv_a, wvv_a, srcv_b, dstv_b, wvv_b,
            src_sel, dst_sel, w_sel,
            rows_v, src_idx, acc, sem_a, sem_b):
    cid = lax.axis_index("c")
    sid = lax.axis_index("s")
    w = sid * NC + cid
    is_last = w == NW - 1
    wlo = w * Z
    nrows = jnp.where(is_last, ZLAST, Z)
    whi = wlo + nrows
    zero16 = jnp.zeros((L,), jnp.float32)

    @pl.loop(0, ZLAST)
    def _(r):
        for k in range(D // L):
            acc[r, pl.ds(k * L, L)] = zero16

    def drain(dbase):
        for k in range(C // L):
            src_idx[pl.ds(k * L, L)] = src_sel[pl.ds(dbase + k * L, L)]
        pltpu.sync_copy(h_hbm.at[src_idx], rows_v)

        @plsc.parallel_loop(0, C, unroll=2)
        def _(r):
            pos16 = jnp.full((L,), dbase + r, jnp.int32)
            wb = plsc.load_gather(w_sel, [pos16])
            dl = plsc.load_gather(dst_sel, [pos16])[0]
            for k in range(D // L):
                sl = pl.ds(k * L, L)
                plsc.addupdate(acc.at[dl, sl], rows_v[r, sl] * wb)

    def start_fetch(ch, sv, dv, wv, sem):
        e0 = ch * EC
        pltpu.async_copy(ei_hbm.at[pl.ds(e0, EC)], sv, sem)
        pltpu.async_copy(ei_hbm.at[pl.ds(N_EDGES + e0, EC)], dv, sem)
        pltpu.async_copy(ew_hbm.at[pl.ds(e0, EC)], wv, sem)

    def wait_fetch(sv, dv, wv, sem):
        pltpu.make_async_copy(ei_hbm.at[pl.ds(0, EC)], sv, sem).wait()
        pltpu.make_async_copy(ei_hbm.at[pl.ds(0, EC)], dv, sem).wait()
        pltpu.make_async_copy(ew_hbm.at[pl.ds(0, EC)], wv, sem).wait()

    def compact_drain(srcv, dstv, wvv, cnt):
        def inner(i, cnt):
            dvec = dstv[pl.ds(i * L, L)]
            svec = srcv[pl.ds(i * L, L)]
            wvec = wvv[pl.ds(i * L, L)]
            m = (dvec >= wlo) & (dvec < whi)
            plsc.store_compressed(dst_sel.at[pl.ds(cnt, L)], dvec - wlo, mask=m)
            plsc.store_compressed(src_sel.at[pl.ds(cnt, L)], svec, mask=m)
            plsc.store_compressed(w_sel.at[pl.ds(cnt, L)], wvec, mask=m)
            return cnt + jnp.sum(m.astype(jnp.int32))

        cnt = lax.fori_loop(0, EC // L, inner, cnt)

        nd = cnt // C

        def dr(d, carry):
            drain(d * C)
            return carry

        lax.fori_loop(0, nd, dr, jnp.int32(0))
        off = nd * C
        for k in range(C // L):
            src_sel[pl.ds(k * L, L)] = src_sel[pl.ds(off + k * L, L)]
            dst_sel[pl.ds(k * L, L)] = dst_sel[pl.ds(off + k * L, L)]
            w_sel[pl.ds(k * L, L)] = w_sel[pl.ds(off + k * L, L)]
        return cnt - off

    start_fetch(0, srcv_a, dstv_a, wvv_a, sem_a)

    def two_chunks(p, cnt):
        ch = p * 2
        start_fetch(ch + 1, srcv_b, dstv_b, wvv_b, sem_b)
        wait_fetch(srcv_a, dstv_a, wvv_a, sem_a)
        cnt = compact_drain(srcv_a, dstv_a, wvv_a, cnt)

        @pl.when(ch + 2 < NCHUNKS)
        def _():
            start_fetch(ch + 2, srcv_a, dstv_a, wvv_a, sem_a)

        wait_fetch(srcv_b, dstv_b, wvv_b, sem_b)
        cnt = compact_drain(srcv_b, dstv_b, wvv_b, cnt)
        return cnt

    cnt = lax.fori_loop(0, NCHUNKS // 2, two_chunks, jnp.int32(0))

    base = (cnt // L) * L
    lane = lax.iota(jnp.int32, L)
    for t in range(C // L + 1):
        off = base + t * L
        keep = (off + lane) < cnt
        src_sel[pl.ds(off, L)] = jnp.where(keep, src_sel[pl.ds(off, L)], 0)
        dst_sel[pl.ds(off, L)] = jnp.where(keep, dst_sel[pl.ds(off, L)], 0)
        w_sel[pl.ds(off, L)] = jnp.where(keep, w_sel[pl.ds(off, L)], 0.0)

    @pl.when(cnt > 0)
    def _():
        drain(0)

    @pl.when(jnp.logical_not(is_last))
    def _():
        pltpu.sync_copy(acc.at[pl.ds(0, Z)], agg_hbm.at[pl.ds(wlo, Z)])

    @pl.when(is_last)
    def _():
        pltpu.sync_copy(acc.at[pl.ds(0, ZLAST)],
                        agg_hbm.at[pl.ds((NW - 1) * Z, ZLAST)])


def _tc_body(h_ref, agg_ref, w_ref, b_ref, o_ref):
    u = jnp.dot(agg_ref[...], w_ref[...], preferred_element_type=jnp.float32)
    u = u + b_ref[...]
    g = 0.5 * u * (1.0 + lax.erf(u * 0.7071067811865476))
    o_ref[...] = h_ref[...] + g


_R = 2000


def _tc_update(h, agg, W, b2):
    return pl.pallas_call(
        _tc_body,
        grid=(N_NODES // _R,),
        in_specs=[
            pl.BlockSpec((_R, D), lambda i: (i, 0)),
            pl.BlockSpec((_R, D), lambda i: (i, 0)),
            pl.BlockSpec((D, D), lambda i: (0, 0)),
            pl.BlockSpec((1, D), lambda i: (0, 0)),
        ],
        out_specs=pl.BlockSpec((_R, D), lambda i: (i, 0)),
        out_shape=jax.ShapeDtypeStruct((N_NODES, D), jnp.float32),
    )(h, agg, W, b2)


def kernel(h, edge_index, edge_w, W, b):
    agg = _sc_agg(h, edge_index.reshape(2 * N_EDGES), edge_w)
    return _tc_update(h, agg, W, b.reshape(1, D))


# speedup vs baseline: 5.7253x; 1.6881x over previous
import functools

import jax
import jax.numpy as jnp
from jax import lax
from jax.experimental import pallas as pl
from jax.experimental.pallas import tpu as pltpu
from jax.experimental.pallas import tpu_sc as plsc

N_NODES = 10000
N_EDGES = 160000
D = 256
NC = 2
NS = 16
NW = NC * NS
L = 16
Z = 312
ZLAST = N_NODES - Z * (NW - 1)
EC = 1600
NCHUNKS = N_EDGES // EC
C = 48
SELCAP = (C - 1) + EC + 2 * L

_mesh = plsc.VectorSubcoreMesh(
    core_axis_name="c", subcore_axis_name="s", num_cores=NC, num_subcores=NS
)


@functools.partial(
    pl.kernel,
    out_type=jax.ShapeDtypeStruct((N_NODES, D), jnp.float32),
    mesh=_mesh,
    compiler_params=pltpu.CompilerParams(needs_layout_passes=False),
    scratch_types=[
        pltpu.VMEM((EC,), jnp.int32),
        pltpu.VMEM((EC,), jnp.int32),
        pltpu.VMEM((EC,), jnp.float32),
        pltpu.VMEM((EC,), jnp.int32),
        pltpu.VMEM((EC,), jnp.int32),
        pltpu.VMEM((EC,), jnp.float32),
        pltpu.VMEM((SELCAP,), jnp.int32),
        pltpu.VMEM((SELCAP,), jnp.int32),
        pltpu.VMEM((SELCAP,), jnp.float32),
        pltpu.VMEM((C, D), jnp.float32),
        pltpu.VMEM((C, D), jnp.float32),
        pltpu.VMEM((C,), jnp.int32),
        pltpu.VMEM((C,), jnp.int32),
        pltpu.VMEM((C,), jnp.int32),
        pltpu.VMEM((C,), jnp.float32),
        pltpu.VMEM((C,), jnp.int32),
        pltpu.VMEM((C,), jnp.float32),
        pltpu.VMEM((ZLAST, D), jnp.float32),
        pltpu.SemaphoreType.DMA,
        pltpu.SemaphoreType.DMA,
        pltpu.SemaphoreType.DMA,
        pltpu.SemaphoreType.DMA,
    ],
)
def _sc_agg(h_hbm, ei_hbm, ew_hbm, agg_hbm,
            srcv_a, dstv_a, wvv_a, srcv_b, dstv_b, wvv_b,
            src_sel, dst_sel, w_sel,
            rows0, rows1, si0, si1, dp0, wp0, dp1, wp1,
            acc, sem_a, sem_b, sem_g0, sem_g1):
    cid = lax.axis_index("c")
    sid = lax.axis_index("s")
    w = sid * NC + cid
    is_last = w == NW - 1
    wlo = w * Z
    nrows = jnp.where(is_last, ZLAST, Z)
    whi = wlo + nrows
    zero16 = jnp.zeros((L,), jnp.float32)

    @pl.loop(0, ZLAST)
    def _(r):
        for k in range(D // L):
            acc[r, pl.ds(k * L, L)] = zero16

    def issue_impl(dbase, si_x, dp_x, wp_x, rows_x, sem_x):
        for k in range(C // L):
            si_x[pl.ds(k * L, L)] = src_sel[pl.ds(dbase + k * L, L)]
            dp_x[pl.ds(k * L, L)] = dst_sel[pl.ds(dbase + k * L, L)]
            wp_x[pl.ds(k * L, L)] = w_sel[pl.ds(dbase + k * L, L)]
        pltpu.async_copy(h_hbm.at[si_x], rows_x, sem_x)

    def complete_impl(si_x, dp_x, wp_x, rows_x, sem_x):
        pltpu.make_async_copy(h_hbm.at[si_x], rows_x, sem_x).wait()

        @plsc.parallel_loop(0, C, unroll=2)
        def _(r):
            pos16 = jnp.full((L,), r, jnp.int32)
            wb = plsc.load_gather(wp_x, [pos16])
            dl = plsc.load_gather(dp_x, [pos16])[0]
            for k in range(D // L):
                sl = pl.ds(k * L, L)
                plsc.addupdate(acc.at[dl, sl], rows_x[r, sl] * wb)

    def issue_b(dbase, qi):
        @pl.when(qi % 2 == 0)
        def _():
            issue_impl(dbase, si0, dp0, wp0, rows0, sem_g0)

        @pl.when(qi % 2 == 1)
        def _():
            issue_impl(dbase, si1, dp1, wp1, rows1, sem_g1)

    def complete_b(qc):
        @pl.when(qc % 2 == 0)
        def _():
            complete_impl(si0, dp0, wp0, rows0, sem_g0)

        @pl.when(qc % 2 == 1)
        def _():
            complete_impl(si1, dp1, wp1, rows1, sem_g1)

    def start_fetch(ch, sv, dv, wv, sem):
        e0 = ch * EC
        pltpu.async_copy(ei_hbm.at[pl.ds(e0, EC)], sv, sem)
        pltpu.async_copy(ei_hbm.at[pl.ds(N_EDGES + e0, EC)], dv, sem)
        pltpu.async_copy(ew_hbm.at[pl.ds(e0, EC)], wv, sem)

    def wait_fetch(sv, dv, wv, sem):
        pltpu.make_async_copy(ei_hbm.at[pl.ds(0, EC)], sv, sem).wait()
        pltpu.make_async_copy(ei_hbm.at[pl.ds(0, EC)], dv, sem).wait()
        pltpu.make_async_copy(ew_hbm.at[pl.ds(0, EC)], wv, sem).wait()

    def compact_drain(srcv, dstv, wvv, cnt, qi, qc):
        def inner(i, cnt):
            dvec = dstv[pl.ds(i * L, L)]
            svec = srcv[pl.ds(i * L, L)]
            wvec = wvv[pl.ds(i * L, L)]
            m = (dvec >= wlo) & (dvec < whi)
            plsc.store_compressed(dst_sel.at[pl.ds(cnt, L)], dvec - wlo, mask=m)
            plsc.store_compressed(src_sel.at[pl.ds(cnt, L)], svec, mask=m)
            plsc.store_compressed(w_sel.at[pl.ds(cnt, L)], wvec, mask=m)
            return cnt + jnp.sum(m.astype(jnp.int32))

        cnt = plsc.parallel_loop(0, EC // L, unroll=4, carry=cnt)(inner)
        nd = cnt // C

        def dr(d, st):
            qi, qc = st
            issue_b(d * C, qi)
            qi = qi + 1
            two_out = qi - qc >= 2

            @pl.when(two_out)
            def _():
                complete_b(qc)

            qc = jnp.where(two_out, qc + 1, qc)
            return (qi, qc)

        qi, qc = lax.fori_loop(0, nd, dr, (qi, qc))
        off = nd * C
        for k in range(C // L):
            src_sel[pl.ds(k * L, L)] = src_sel[pl.ds(off + k * L, L)]
            dst_sel[pl.ds(k * L, L)] = dst_sel[pl.ds(off + k * L, L)]
            w_sel[pl.ds(k * L, L)] = w_sel[pl.ds(off + k * L, L)]
        return cnt - off, qi, qc

    start_fetch(0, srcv_a, dstv_a, wvv_a, sem_a)

    def two_chunks(p, st):
        ch = p * 2
        cnt, qi, qc = st
        start_fetch(ch + 1, srcv_b, dstv_b, wvv_b, sem_b)
        wait_fetch(srcv_a, dstv_a, wvv_a, sem_a)
        cnt, qi, qc = compact_drain(srcv_a, dstv_a, wvv_a, cnt, qi, qc)

        @pl.when(ch + 2 < NCHUNKS)
        def _():
            start_fetch(ch + 2, srcv_a, dstv_a, wvv_a, sem_a)

        wait_fetch(srcv_b, dstv_b, wvv_b, sem_b)
        cnt, qi, qc = compact_drain(srcv_b, dstv_b, wvv_b, cnt, qi, qc)
        return (cnt, qi, qc)

    cnt, qi, qc = lax.fori_loop(
        0, NCHUNKS // 2, two_chunks,
        (jnp.int32(0), jnp.int32(0), jnp.int32(0)))

    base = (cnt // L) * L
    lane = lax.iota(jnp.int32, L)
    for t in range(C // L + 1):
        off = base + t * L
        keep = (off + lane) < cnt
        src_sel[pl.ds(off, L)] = jnp.where(keep, src_sel[pl.ds(off, L)], 0)
        dst_sel[pl.ds(off, L)] = jnp.where(keep, dst_sel[pl.ds(off, L)], 0)
        w_sel[pl.ds(off, L)] = jnp.where(keep, w_sel[pl.ds(off, L)], 0.0)

    @pl.when(cnt > 0)
    def _():
        issue_b(0, qi)

    qi = qi + (cnt > 0).astype(jnp.int32)
    for _ in range(2):
        @pl.when(qc < qi)
        def _():
            complete_b(qc)

        qc = jnp.where(qc < qi, qc + 1, qc)

    @pl.when(jnp.logical_not(is_last))
    def _():
        pltpu.sync_copy(acc.at[pl.ds(0, Z)], agg_hbm.at[pl.ds(wlo, Z)])

    @pl.when(is_last)
    def _():
        pltpu.sync_copy(acc.at[pl.ds(0, ZLAST)],
                        agg_hbm.at[pl.ds((NW - 1) * Z, ZLAST)])


def _tc_body(h_ref, agg_ref, w_ref, b_ref, o_ref):
    u = jnp.dot(agg_ref[...], w_ref[...], preferred_element_type=jnp.float32)
    u = u + b_ref[...]
    g = 0.5 * u * (1.0 + lax.erf(u * 0.7071067811865476))
    o_ref[...] = h_ref[...] + g


_R = 2000


def _tc_update(h, agg, W, b2):
    return pl.pallas_call(
        _tc_body,
        grid=(N_NODES // _R,),
        in_specs=[
            pl.BlockSpec((_R, D), lambda i: (i, 0)),
            pl.BlockSpec((_R, D), lambda i: (i, 0)),
            pl.BlockSpec((D, D), lambda i: (0, 0)),
            pl.BlockSpec((1, D), lambda i: (0, 0)),
        ],
        out_specs=pl.BlockSpec((_R, D), lambda i: (i, 0)),
        out_shape=jax.ShapeDtypeStruct((N_NODES, D), jnp.float32),
    )(h, agg, W, b2)


def kernel(h, edge_index, edge_w, W, b):
    agg = _sc_agg(h, edge_index.reshape(2 * N_EDGES), edge_w)
    return _tc_update(h, agg, W, b.reshape(1, D))
